# Initial kernel scaffold; baseline (speedup 1.0000x reference)
import jax
import jax.numpy as jnp
from jax.experimental import pallas as pl


def kernel(num_mentions_total, num_mentions_named, num_mentions_nominal, num_mentions_pronominal, W_total, W_named, W_nominal, W_pronominal):
    raise NotImplementedError("write your pallas kernel here")


# speedup vs baseline: 4.2405x; 4.2405x over previous
import functools

import jax
import jax.numpy as jnp
from jax import lax
from jax.experimental import pallas as pl
from jax.experimental.pallas import tpu as pltpu
from jax.experimental.pallas import tpu_sc as plsc

VOCAB = 100000
EDIM = 64
B = 4096
L = 200
NUM_TABLES = 4

NC = 2
NS = 16
NW = NC * NS
ROWS = B * L * NUM_TABLES
RPW = ROWS // NW
CH = 512
NCH = RPW // CH


def _make_gather():
    mesh = plsc.VectorSubcoreMesh(core_axis_name="c", subcore_axis_name="s")

    @functools.partial(
        pl.kernel,
        mesh=mesh,
        compiler_params=pltpu.CompilerParams(use_tc_tiling_on_sc=False),
        out_type=jax.ShapeDtypeStruct((ROWS, EDIM), jnp.float32),
        scratch_types=[
            pltpu.VMEM((CH,), jnp.int32),
            pltpu.VMEM((CH, EDIM), jnp.float32),
            pltpu.SemaphoreType.DMA,
        ],
    )
    def gather_kernel(table_hbm, idx_hbm, out_hbm, idx_v, rows_v, sem):
        wid = lax.axis_index("s") * NC + lax.axis_index("c")
        w0 = wid * RPW

        def body(i, carry):
            base = w0 + i * CH
            pltpu.sync_copy(idx_hbm.at[pl.ds(base, CH)], idx_v)
            pltpu.async_copy(table_hbm.at[idx_v], rows_v, sem).wait()
            pltpu.sync_copy(rows_v, out_hbm.at[pl.ds(base, CH)])
            return carry

        lax.fori_loop(0, NCH, body, 0)

    return gather_kernel


_gather = _make_gather()


def kernel(num_mentions_total, num_mentions_named, num_mentions_nominal,
           num_mentions_pronominal, W_total, W_named, W_nominal,
           W_pronominal):
    table = jnp.concatenate([W_total, W_named, W_nominal, W_pronominal],
                            axis=0)
    idx = jnp.stack([num_mentions_total, num_mentions_named,
                     num_mentions_nominal, num_mentions_pronominal],
                    axis=-1).astype(jnp.int32)
    idx = idx + jnp.arange(NUM_TABLES, dtype=jnp.int32) * VOCAB
    out = _gather(table, idx.reshape(-1))
    return out.reshape(B, L, NUM_TABLES * EDIM)


# speedup vs baseline: 4.5541x; 1.0739x over previous
import functools

import jax
import jax.numpy as jnp
from jax import lax
from jax.experimental import pallas as pl
from jax.experimental.pallas import tpu as pltpu
from jax.experimental.pallas import tpu_sc as plsc

VOCAB = 100000
EDIM = 64
B = 4096
L = 200
NUM_TABLES = 4

NC = 2
NS = 16
NW = NC * NS
ROWS = B * L * NUM_TABLES
RPW = ROWS // NW
CH = 800
NCH = RPW // CH


def _make_gather():
    mesh = plsc.VectorSubcoreMesh(core_axis_name="c", subcore_axis_name="s")

    @functools.partial(
        pl.kernel,
        mesh=mesh,
        compiler_params=pltpu.CompilerParams(use_tc_tiling_on_sc=False),
        out_type=jax.ShapeDtypeStruct((ROWS, EDIM), jnp.float32),
        scratch_types=[
            pltpu.VMEM((CH,), jnp.int32),
            pltpu.VMEM((CH,), jnp.int32),
            pltpu.VMEM((CH, EDIM), jnp.float32),
            pltpu.VMEM((CH, EDIM), jnp.float32),
            pltpu.SemaphoreType.DMA,
            pltpu.SemaphoreType.DMA,
            pltpu.SemaphoreType.DMA,
        ],
    )
    def gather_kernel(table_hbm, idx_hbm, out_hbm,
                      idx_v0, idx_v1, rows_v0, rows_v1, isem, gsem, wsem):
        wid = lax.axis_index("s") * NC + lax.axis_index("c")
        w0 = wid * RPW

        def issue_idx(i, dst):
            return pltpu.async_copy(idx_hbm.at[pl.ds(w0 + i * CH, CH)],
                                    dst, isem)

        def wait_idx():
            pltpu.make_async_copy(idx_hbm.at[pl.ds(w0, CH)], idx_v0,
                                  isem).wait()

        def issue_gather(idx_v, rows_v):
            return pltpu.async_copy(table_hbm.at[idx_v], rows_v, gsem)

        def wait_gather(idx_v, rows_v):
            pltpu.make_async_copy(table_hbm.at[idx_v], rows_v, gsem).wait()

        def issue_write(i, rows_v):
            return pltpu.async_copy(rows_v,
                                    out_hbm.at[pl.ds(w0 + i * CH, CH)], wsem)

        def wait_write():
            pltpu.make_async_copy(rows_v0, out_hbm.at[pl.ds(w0, CH)],
                                  wsem).wait()

        issue_idx(0, idx_v0)
        issue_idx(1, idx_v1)
        wait_idx()
        issue_gather(idx_v0, rows_v0)

        def body(k, carry):
            i0 = 2 * k
            i1 = i0 + 1
            wait_gather(idx_v0, rows_v0)
            issue_write(i0, rows_v0)

            @pl.when(k > 0)
            def _():
                wait_write()

            wait_idx()
            issue_gather(idx_v1, rows_v1)

            @pl.when(i0 + 2 < NCH)
            def _():
                issue_idx(i0 + 2, idx_v0)

            wait_gather(idx_v1, rows_v1)
            issue_write(i1, rows_v1)
            wait_write()

            @pl.when(i1 + 1 < NCH)
            def _():
                wait_idx()
                issue_gather(idx_v0, rows_v0)

            @pl.when(i1 + 2 < NCH)
            def _():
                issue_idx(i1 + 2, idx_v1)

            return carry

        lax.fori_loop(0, NCH // 2, body, 0)
        wait_write()

    return gather_kernel


_gather = _make_gather()


def kernel(num_mentions_total, num_mentions_named, num_mentions_nominal,
           num_mentions_pronominal, W_total, W_named, W_nominal,
           W_pronominal):
    table = jnp.concatenate([W_total, W_named, W_nominal, W_pronominal],
                            axis=0)
    idx = jnp.stack([num_mentions_total, num_mentions_named,
                     num_mentions_nominal, num_mentions_pronominal],
                    axis=-1).astype(jnp.int32)
    idx = idx + jnp.arange(NUM_TABLES, dtype=jnp.int32) * VOCAB
    out = _gather(table, idx.reshape(-1))
    return out.reshape(B, L, NUM_TABLES * EDIM)


# speedup vs baseline: 4.5654x; 1.0025x over previous
import functools

import jax
import jax.numpy as jnp
from jax import lax
from jax.experimental import pallas as pl
from jax.experimental.pallas import tpu as pltpu
from jax.experimental.pallas import tpu_sc as plsc

VOCAB = 100000
EDIM = 64
B = 4096
L = 200
NUM_TABLES = 4

NC = 2
NS = 16
NW = NC * NS
ROWS = B * L * NUM_TABLES
RPW = ROWS // NW
CH = 800
NCH = RPW // CH


def _make_gather():
    mesh = plsc.VectorSubcoreMesh(core_axis_name="c", subcore_axis_name="s")

    @functools.partial(
        pl.kernel,
        mesh=mesh,
        compiler_params=pltpu.CompilerParams(use_tc_tiling_on_sc=False),
        out_type=jax.ShapeDtypeStruct((ROWS, EDIM), jnp.float32),
        scratch_types=[
            pltpu.VMEM((CH,), jnp.int32),
            pltpu.VMEM((CH,), jnp.int32),
            pltpu.VMEM((CH, EDIM), jnp.float32),
            pltpu.VMEM((CH, EDIM), jnp.float32),
            pltpu.SemaphoreType.DMA,
            pltpu.SemaphoreType.DMA,
            pltpu.SemaphoreType.DMA,
            pltpu.SemaphoreType.DMA,
        ],
    )
    def gather_kernel(table_hbm, idx_hbm, out_hbm,
                      idx_v0, idx_v1, rows_v0, rows_v1,
                      isem, gsem, wsem0, wsem1):
        wid = lax.axis_index("s") * NC + lax.axis_index("c")
        w0 = wid * RPW

        def issue_idx(i, dst):
            return pltpu.async_copy(idx_hbm.at[pl.ds(w0 + i * CH, CH)],
                                    dst, isem)

        def wait_idx():
            pltpu.make_async_copy(idx_hbm.at[pl.ds(w0, CH)], idx_v0,
                                  isem).wait()

        def issue_gather(idx_v, rows_v):
            return pltpu.async_copy(table_hbm.at[idx_v], rows_v, gsem)

        def wait_gather(idx_v, rows_v):
            pltpu.make_async_copy(table_hbm.at[idx_v], rows_v, gsem).wait()

        def issue_write(i, rows_v, wsem):
            return pltpu.async_copy(rows_v,
                                    out_hbm.at[pl.ds(w0 + i * CH, CH)], wsem)

        def wait_write(rows_v, wsem):
            pltpu.make_async_copy(rows_v, out_hbm.at[pl.ds(w0, CH)],
                                  wsem).wait()

        issue_idx(0, idx_v0)
        issue_idx(1, idx_v1)
        wait_idx()
        issue_gather(idx_v0, rows_v0)

        def body(k, carry):
            i0 = 2 * k
            i1 = i0 + 1
            wait_gather(idx_v0, rows_v0)
            issue_write(i0, rows_v0, wsem0)

            @pl.when(k > 0)
            def _():
                wait_write(rows_v1, wsem1)

            wait_idx()
            issue_gather(idx_v1, rows_v1)

            @pl.when(i0 + 2 < NCH)
            def _():
                issue_idx(i0 + 2, idx_v0)

            wait_gather(idx_v1, rows_v1)
            issue_write(i1, rows_v1, wsem1)
            wait_write(rows_v0, wsem0)

            @pl.when(i1 + 1 < NCH)
            def _():
                wait_idx()
                issue_gather(idx_v0, rows_v0)

            @pl.when(i1 + 2 < NCH)
            def _():
                issue_idx(i1 + 2, idx_v1)

            return carry

        lax.fori_loop(0, NCH // 2, body, 0)
        wait_write(rows_v1, wsem1)

    return gather_kernel


_gather = _make_gather()


def kernel(num_mentions_total, num_mentions_named, num_mentions_nominal,
           num_mentions_pronominal, W_total, W_named, W_nominal,
           W_pronominal):
    table = jnp.concatenate([W_total, W_named, W_nominal, W_pronominal],
                            axis=0)
    idx = jnp.stack([num_mentions_total, num_mentions_named,
                     num_mentions_nominal, num_mentions_pronominal],
                    axis=-1).astype(jnp.int32)
    idx = idx + jnp.arange(NUM_TABLES, dtype=jnp.int32) * VOCAB
    out = _gather(table, idx.reshape(-1))
    return out.reshape(B, L, NUM_TABLES * EDIM)


# speedup vs baseline: 7.8405x; 1.7174x over previous
import functools

import jax
import jax.numpy as jnp
from jax import lax
from jax.experimental import pallas as pl
from jax.experimental.pallas import tpu as pltpu
from jax.experimental.pallas import tpu_sc as plsc

VOCAB = 100000
EDIM = 64
B = 4096
L = 200
NUM_TABLES = 4

NC = 2
NS = 16
NW = NC * NS
PAIRS = B * L
PPW = PAIRS // NW
PCH = 200
NCH = PPW // PCH


def _make_gather():
    mesh = plsc.VectorSubcoreMesh(core_axis_name="c", subcore_axis_name="s")

    @functools.partial(
        pl.kernel,
        mesh=mesh,
        compiler_params=pltpu.CompilerParams(use_tc_tiling_on_sc=False),
        out_type=jax.ShapeDtypeStruct((PAIRS, NUM_TABLES, EDIM),
                                      jnp.float32),
        scratch_types=[
            pltpu.VMEM((NUM_TABLES, PCH), jnp.int32),
            pltpu.VMEM((NUM_TABLES, PCH), jnp.int32),
            pltpu.VMEM((NUM_TABLES, PCH, EDIM), jnp.float32),
            pltpu.VMEM((NUM_TABLES, PCH, EDIM), jnp.float32),
            pltpu.SemaphoreType.DMA,
            pltpu.SemaphoreType.DMA,
            pltpu.SemaphoreType.DMA,
            pltpu.SemaphoreType.DMA,
        ],
    )
    def gather_kernel(t0, t1, t2, t3, i0_hbm, i1_hbm, i2_hbm, i3_hbm,
                      out_hbm, idx_v0, idx_v1, rows_v0, rows_v1,
                      isem, gsem, wsem0, wsem1):
        tables = (t0, t1, t2, t3)
        idxs = (i0_hbm, i1_hbm, i2_hbm, i3_hbm)
        wid = lax.axis_index("s") * NC + lax.axis_index("c")
        w0 = wid * PPW

        def issue_idx(i, dst):
            for t in range(NUM_TABLES):
                pltpu.async_copy(idxs[t].at[pl.ds(w0 + i * PCH, PCH)],
                                 dst.at[t], isem)

        def wait_idx():
            for t in range(NUM_TABLES):
                pltpu.make_async_copy(idxs[t].at[pl.ds(w0, PCH)],
                                      idx_v0.at[t], isem).wait()

        def issue_gather(idx_v, rows_v):
            for t in range(NUM_TABLES):
                pltpu.async_copy(tables[t].at[idx_v.at[t]],
                                 rows_v.at[t], gsem)

        def wait_gather(idx_v, rows_v):
            for t in range(NUM_TABLES):
                pltpu.make_async_copy(tables[t].at[idx_v.at[t]],
                                      rows_v.at[t], gsem).wait()

        def issue_write(i, rows_v, wsem):
            for t in range(NUM_TABLES):
                pltpu.async_copy(rows_v.at[t],
                                 out_hbm.at[pl.ds(w0 + i * PCH, PCH), t],
                                 wsem)

        def wait_write(rows_v, wsem):
            for t in range(NUM_TABLES):
                pltpu.make_async_copy(rows_v.at[t],
                                      out_hbm.at[pl.ds(w0, PCH), t],
                                      wsem).wait()

        issue_idx(0, idx_v0)
        issue_idx(1, idx_v1)
        wait_idx()
        issue_gather(idx_v0, rows_v0)

        def body(k, carry):
            c0 = 2 * k
            c1 = c0 + 1
            wait_gather(idx_v0, rows_v0)
            issue_write(c0, rows_v0, wsem0)

            @pl.when(k > 0)
            def _():
                wait_write(rows_v1, wsem1)

            wait_idx()
            issue_gather(idx_v1, rows_v1)

            @pl.when(c0 + 2 < NCH)
            def _():
                issue_idx(c0 + 2, idx_v0)

            wait_gather(idx_v1, rows_v1)
            issue_write(c1, rows_v1, wsem1)
            wait_write(rows_v0, wsem0)

            @pl.when(c1 + 1 < NCH)
            def _():
                wait_idx()
                issue_gather(idx_v0, rows_v0)

            @pl.when(c1 + 2 < NCH)
            def _():
                issue_idx(c1 + 2, idx_v1)

            return carry

        lax.fori_loop(0, NCH // 2, body, 0)
        wait_write(rows_v1, wsem1)

    return gather_kernel


_gather = _make_gather()


def kernel(num_mentions_total, num_mentions_named, num_mentions_nominal,
           num_mentions_pronominal, W_total, W_named, W_nominal,
           W_pronominal):
    out = _gather(W_total, W_named, W_nominal, W_pronominal,
                  num_mentions_total.astype(jnp.int32).reshape(-1),
                  num_mentions_named.astype(jnp.int32).reshape(-1),
                  num_mentions_nominal.astype(jnp.int32).reshape(-1),
                  num_mentions_pronominal.astype(jnp.int32).reshape(-1))
    return out.reshape(B, L, NUM_TABLES * EDIM)


# speedup vs baseline: 7.8480x; 1.0010x over previous
import functools

import jax
import jax.numpy as jnp
from jax import lax
from jax.experimental import pallas as pl
from jax.experimental.pallas import tpu as pltpu
from jax.experimental.pallas import tpu_sc as plsc

VOCAB = 100000
EDIM = 64
B = 4096
L = 200
NUM_TABLES = 4

NC = 2
NS = 16
NW = NC * NS
PAIRS = B * L
PPW = PAIRS // NW
PCH = 200
NCH = PPW // PCH


def _make_gather():
    mesh = plsc.VectorSubcoreMesh(core_axis_name="c", subcore_axis_name="s",
                                  num_cores=NC)

    @functools.partial(
        pl.kernel,
        mesh=mesh,
        compiler_params=pltpu.CompilerParams(use_tc_tiling_on_sc=False),
        out_type=jax.ShapeDtypeStruct((PAIRS, NUM_TABLES, EDIM),
                                      jnp.float32),
        scratch_types=[
            pltpu.VMEM((NUM_TABLES, PCH), jnp.int32),
            pltpu.VMEM((NUM_TABLES, PCH), jnp.int32),
            pltpu.VMEM((NUM_TABLES, PCH, EDIM), jnp.float32),
            pltpu.VMEM((NUM_TABLES, PCH, EDIM), jnp.float32),
            pltpu.SemaphoreType.DMA,
            pltpu.SemaphoreType.DMA,
            pltpu.SemaphoreType.DMA,
            pltpu.SemaphoreType.DMA,
        ],
    )
    def gather_kernel(t0, t1, t2, t3, i0_hbm, i1_hbm, i2_hbm, i3_hbm,
                      out_hbm, idx_v0, idx_v1, rows_v0, rows_v1,
                      isem, gsem, wsem0, wsem1):
        tables = (t0, t1, t2, t3)
        idxs = (i0_hbm, i1_hbm, i2_hbm, i3_hbm)
        wid = lax.axis_index("s") * NC + lax.axis_index("c")
        w0 = wid * PPW

        def issue_idx(i, dst):
            for t in range(NUM_TABLES):
                pltpu.async_copy(idxs[t].at[pl.ds(w0 + i * PCH, PCH)],
                                 dst.at[t], isem)

        def wait_idx():
            for t in range(NUM_TABLES):
                pltpu.make_async_copy(idxs[t].at[pl.ds(w0, PCH)],
                                      idx_v0.at[t], isem).wait()

        def issue_gather(idx_v, rows_v):
            for t in range(NUM_TABLES):
                pltpu.async_copy(tables[t].at[idx_v.at[t]],
                                 rows_v.at[t], gsem)

        def wait_gather(idx_v, rows_v):
            for t in range(NUM_TABLES):
                pltpu.make_async_copy(tables[t].at[idx_v.at[t]],
                                      rows_v.at[t], gsem).wait()

        def issue_write(i, rows_v, wsem):
            for t in range(NUM_TABLES):
                pltpu.async_copy(rows_v.at[t],
                                 out_hbm.at[pl.ds(w0 + i * PCH, PCH), t],
                                 wsem)

        def wait_write(rows_v, wsem):
            for t in range(NUM_TABLES):
                pltpu.make_async_copy(rows_v.at[t],
                                      out_hbm.at[pl.ds(w0, PCH), t],
                                      wsem).wait()

        issue_idx(0, idx_v0)
        issue_idx(1, idx_v1)
        wait_idx()
        issue_gather(idx_v0, rows_v0)

        def body(k, carry):
            c0 = 2 * k
            c1 = c0 + 1
            wait_gather(idx_v0, rows_v0)
            issue_write(c0, rows_v0, wsem0)

            @pl.when(k > 0)
            def _():
                wait_write(rows_v1, wsem1)

            wait_idx()
            issue_gather(idx_v1, rows_v1)

            @pl.when(c0 + 2 < NCH)
            def _():
                issue_idx(c0 + 2, idx_v0)

            wait_gather(idx_v1, rows_v1)
            issue_write(c1, rows_v1, wsem1)
            wait_write(rows_v0, wsem0)

            @pl.when(c1 + 1 < NCH)
            def _():
                wait_idx()
                issue_gather(idx_v0, rows_v0)

            @pl.when(c1 + 2 < NCH)
            def _():
                issue_idx(c1 + 2, idx_v1)

            return carry

        lax.fori_loop(0, NCH // 2, body, 0)
        wait_write(rows_v1, wsem1)

    return gather_kernel


_gather = _make_gather()


def kernel(num_mentions_total, num_mentions_named, num_mentions_nominal,
           num_mentions_pronominal, W_total, W_named, W_nominal,
           W_pronominal):
    out = _gather(W_total, W_named, W_nominal, W_pronominal,
                  num_mentions_total.astype(jnp.int32).reshape(-1),
                  num_mentions_named.astype(jnp.int32).reshape(-1),
                  num_mentions_nominal.astype(jnp.int32).reshape(-1),
                  num_mentions_pronominal.astype(jnp.int32).reshape(-1))
    return out.reshape(B, L, NUM_TABLES * EDIM)
